# baseline (speedup 1.0000x reference)
import functools

import jax
import jax.numpy as jnp
from jax import lax
from jax.experimental import pallas as pl
from jax.experimental.pallas import tpu as pltpu
from jax.experimental.pallas import tpu_sc as plsc

CHUNK = 256
SUP_CHUNKS = 5
SUP = CHUNK * SUP_CHUNKS


@functools.cache
def _make_gather(b_total: int, vocab: int, dim: int):
    info = plsc.get_sparse_core_info()
    nw = info.num_cores * info.num_subcores
    b_per_w = b_total // nw
    n_sup = b_per_w // SUP
    assert b_per_w * nw == b_total
    assert n_sup * SUP == b_per_w and n_sup % 2 == 0

    mesh = plsc.VectorSubcoreMesh(core_axis_name="c", subcore_axis_name="s")

    @functools.partial(
        pl.kernel,
        mesh=mesh,
        out_type=jax.ShapeDtypeStruct((b_total, dim), jnp.float32),
        scratch_types=[
            pltpu.VMEM((b_per_w,), jnp.int32),
            pltpu.VMEM((SUP, dim), jnp.float32),
            pltpu.VMEM((SUP, dim), jnp.float32),
            pltpu.SemaphoreType.DMA,
            pltpu.SemaphoreType.DMA,
            pltpu.SemaphoreType.DMA,
            pltpu.SemaphoreType.DMA,
        ],
        compiler_params=pltpu.CompilerParams(use_tc_tiling_on_sc=False),
    )
    def gather_kernel(idx_hbm, table_hbm, out_hbm, idx_v, buf0, buf1,
                      gsem0, gsem1, wsem0, wsem1):
        wid = lax.axis_index("s") * info.num_cores + lax.axis_index("c")
        base = wid * b_per_w
        pltpu.sync_copy(idx_hbm.at[pl.ds(base, b_per_w)], idx_v)

        bufs = (buf0, buf1)
        gsems = (gsem0, gsem1)
        wsems = (wsem0, wsem1)

        def fire(j, buf, gsem):
            for t in range(SUP_CHUNKS):
                pltpu.make_async_copy(
                    table_hbm.at[idx_v.at[pl.ds(j * SUP + t * CHUNK, CHUNK)]],
                    buf.at[pl.ds(t * CHUNK, CHUNK)],
                    gsem,
                ).start()

        def drain(buf, gsem):
            for t in range(SUP_CHUNKS):
                pltpu.make_async_copy(
                    table_hbm.at[idx_v.at[pl.ds(t * CHUNK, CHUNK)]],
                    buf.at[pl.ds(t * CHUNK, CHUNK)],
                    gsem,
                ).wait()

        def wait_wb(buf, wsem):
            pltpu.make_async_copy(
                buf, out_hbm.at[pl.ds(base, SUP)], wsem
            ).wait()

        fire(0, buf0, gsem0)

        def pair(i, carry):
            for parity in range(2):
                j = 2 * i + parity
                cur, oth = bufs[parity], bufs[1 - parity]
                gcur, goth = gsems[parity], gsems[1 - parity]
                wcur, woth = wsems[parity], wsems[1 - parity]

                @pl.when(j + 1 < n_sup)
                def _():
                    @pl.when(j >= 1)
                    def _():
                        wait_wb(oth, woth)

                    fire(j + 1, oth, goth)

                drain(cur, gcur)
                pltpu.make_async_copy(
                    cur, out_hbm.at[pl.ds(base + j * SUP, SUP)], wcur
                ).start()
            return carry

        lax.fori_loop(0, n_sup // 2, pair, 0)
        wait_wb(buf0, wsem0)
        wait_wb(buf1, wsem1)

    return gather_kernel


def kernel(x, word_vectors):
    b, h = x.shape
    vocab, dim = word_vectors.shape
    idx = x.reshape(b * h).astype(jnp.int32)
    out = _make_gather(b * h, vocab, dim)(idx, word_vectors)
    return out.reshape(b, h, dim)


# speedup vs baseline: 1.6237x; 1.6237x over previous
import functools

import jax
import jax.numpy as jnp
from jax import lax
from jax.experimental import pallas as pl
from jax.experimental.pallas import tpu as pltpu
from jax.experimental.pallas import tpu_sc as plsc

SUP_B = 16
HIST_PAD = 56


@functools.cache
def _make_gather(batch: int, hist: int, vocab: int, dim: int):
    info = plsc.get_sparse_core_info()
    nw = info.num_cores * info.num_subcores
    nb = batch // nw
    sup_b = SUP_B
    n_sup = nb // sup_b
    assert nb * nw == batch
    assert n_sup * sup_b == nb and n_sup % 2 == 0

    mesh = plsc.VectorSubcoreMesh(core_axis_name="c", subcore_axis_name="s")

    @functools.partial(
        pl.kernel,
        mesh=mesh,
        out_type=jax.ShapeDtypeStruct((batch, hist, dim), jnp.float32),
        scratch_types=[
            pltpu.VMEM((nb * HIST_PAD,), jnp.int32),
            pltpu.VMEM((sup_b, hist, dim), jnp.float32),
            pltpu.VMEM((sup_b, hist, dim), jnp.float32),
            pltpu.SemaphoreType.DMA,
            pltpu.SemaphoreType.DMA,
            pltpu.SemaphoreType.DMA,
            pltpu.SemaphoreType.DMA,
        ],
        compiler_params=pltpu.CompilerParams(use_tc_tiling_on_sc=False),
    )
    def gather_kernel(x_hbm, table_hbm, out_hbm, idx_v, buf0, buf1,
                      gsem0, gsem1, wsem0, wsem1):
        wid = lax.axis_index("s") * info.num_cores + lax.axis_index("c")
        batch0 = wid * nb
        pltpu.sync_copy(x_hbm.at[wid], idx_v)

        bufs = (buf0, buf1)
        gsems = (gsem0, gsem1)
        wsems = (wsem0, wsem1)

        def fire(j, buf, gsem):
            for t in range(sup_b):
                pltpu.make_async_copy(
                    table_hbm.at[
                        idx_v.at[pl.ds((j * sup_b + t) * HIST_PAD, hist)]
                    ],
                    buf.at[t],
                    gsem,
                ).start()

        def drain(buf, gsem):
            for t in range(sup_b):
                pltpu.make_async_copy(
                    table_hbm.at[idx_v.at[pl.ds(0, hist)]],
                    buf.at[t],
                    gsem,
                ).wait()

        def wb_copy(j, buf, wsem):
            return pltpu.make_async_copy(
                buf,
                out_hbm.at[pl.ds(batch0 + j * sup_b, sup_b)],
                wsem,
            )

        fire(0, buf0, gsem0)

        def pair(i, carry):
            for parity in range(2):
                j = 2 * i + parity
                cur, oth = bufs[parity], bufs[1 - parity]
                gcur, goth = gsems[parity], gsems[1 - parity]
                wcur, woth = wsems[parity], wsems[1 - parity]

                @pl.when(j + 1 < n_sup)
                def _():
                    @pl.when(j >= 1)
                    def _():
                        wb_copy(0, oth, woth).wait()

                    fire(j + 1, oth, goth)

                drain(cur, gcur)
                wb_copy(j, cur, wcur).start()
            return carry

        lax.fori_loop(0, n_sup // 2, pair, 0)
        wb_copy(0, buf0, wsem0).wait()
        wb_copy(0, buf1, wsem1).wait()

    return gather_kernel


def kernel(x, word_vectors):
    b, h = x.shape
    vocab, dim = word_vectors.shape
    info = plsc.get_sparse_core_info()
    nw = info.num_cores * info.num_subcores
    x_pad = jnp.pad(x.astype(jnp.int32), ((0, 0), (0, HIST_PAD - h)))
    x_by_w = x_pad.reshape(nw, (b // nw) * HIST_PAD)
    return _make_gather(b, h, vocab, dim)(x_by_w, word_vectors)
